# baseline (speedup 1.0000x reference)
import functools

import jax
import jax.numpy as jnp
from jax import lax
from jax.experimental import pallas as pl
from jax.experimental.pallas import tpu as pltpu
from jax.experimental.pallas import tpu_sc as plsc

_BOHR = 0.52917721067
_DAMP = 0.7
_NC = 2
_NS = 16
_NW = _NC * _NS
_C = 2048
_G = _C // 128

_N_NODES = 100000
_N_EDGES = 6400000
_NCHUNKS = _N_EDGES // _C
_TRIPS = -(-_NCHUNKS // _NW)


def _pv_body(pol_ref, out_ref):
    p = pol_ref[...] * (1.0 / _BOHR**3)
    out_ref[...] = jnp.exp(jnp.log(p) * (-1.0 / 6.0)) * (_BOHR**-0.5)


def _sum_body(p_ref, out_ref):
    out_ref[...] = p_ref[0] + p_ref[1]


def _sc_body(src_hbm, dst_hbm, dist_hbm, vec_hbm, pv_hbm, q_hbm, zeros_hbm,
             out_hbm, acc, idxs, idxd, db, vb, psb, pdb, qdb,
             is0, is1, is2, exb, eyb, ezb, sidx, sdat, sgat, ssc):
    cid = lax.axis_index("c")
    sid = lax.axis_index("s")
    wid = sid * _NC + cid

    @pl.when(sid == 0)
    def _zero():
        pltpu.sync_copy(zeros_hbm, acc)

    plsc.subcore_barrier()

    io = lax.iota(jnp.int32, 16)
    io3 = io * 3

    def chunk_body(t, _):
        ch = wid + t * _NW

        @pl.when(ch < _NCHUNKS)
        def _do_chunk():
            base = ch * _C
            grp = ch * _G
            pltpu.async_copy(src_hbm.at[pl.ds(grp, _G)], idxs, sidx)
            pltpu.async_copy(dst_hbm.at[pl.ds(grp, _G)], idxd, sidx)
            pltpu.async_copy(dist_hbm.at[pl.ds(base, _C)], db, sdat)
            pltpu.async_copy(vec_hbm.at[pl.ds(base * 3, _C * 3)], vb, sdat)
            pltpu.make_async_copy(src_hbm.at[pl.ds(grp, _G)], idxs,
                                  sidx).wait()
            pltpu.make_async_copy(dst_hbm.at[pl.ds(grp, _G)], idxd,
                                  sidx).wait()
            for j in range(_G):
                pltpu.async_copy(pv_hbm.at[idxs.at[j]],
                                 psb.at[pl.ds(j * 128, 128)], sgat)
                pltpu.async_copy(pv_hbm.at[idxd.at[j]],
                                 pdb.at[pl.ds(j * 128, 128)], sgat)
                pltpu.async_copy(q_hbm.at[idxd.at[j]],
                                 qdb.at[pl.ds(j * 128, 128)], sgat)
            pltpu.make_async_copy(dist_hbm.at[pl.ds(base, _C)], db,
                                  sdat).wait()
            pltpu.make_async_copy(vec_hbm.at[pl.ds(base * 3, _C * 3)], vb,
                                  sdat).wait()
            for j in range(_G):
                pltpu.make_async_copy(pv_hbm.at[idxs.at[j]],
                                      psb.at[pl.ds(j * 128, 128)],
                                      sgat).wait()
                pltpu.make_async_copy(pv_hbm.at[idxd.at[j]],
                                      pdb.at[pl.ds(j * 128, 128)],
                                      sgat).wait()
                pltpu.make_async_copy(q_hbm.at[idxd.at[j]],
                                      qdb.at[pl.ds(j * 128, 128)],
                                      sgat).wait()

            for j in range(_G):
                srow = idxs.at[j]
                ir0 = is0.at[j]
                ir1 = is1.at[j]
                ir2 = is2.at[j]

                def vbody(w, _, srow=srow, ir0=ir0, ir1=ir1, ir2=ir2, j=j):
                    o = w * 16
                    go = j * 128 + o
                    s16 = srow[pl.ds(o, 16)]
                    s3 = s16 * 3
                    lanes = io + o
                    plsc.store_scatter(ir0, [lanes], s3)
                    plsc.store_scatter(ir1, [lanes], s3 + 1)
                    plsc.store_scatter(ir2, [lanes], s3 + 2)
                    d = db[pl.ds(go, 16)]
                    ps = psb[pl.ds(go, 16)]
                    pd = pdb[pl.ds(go, 16)]
                    qd = qdb[pl.ds(go, 16)]
                    u = d * ps * pd
                    bi = lax.bitcast_convert_type(u, jnp.int32)
                    r = lax.bitcast_convert_type(0x5F3759DF - (bi >> 1),
                                                 jnp.float32)
                    r = r * (1.5 - 0.5 * u * r * r)
                    r = r * (1.5 - 0.5 * u * r * r)
                    r = r * (1.5 - 0.5 * u * r * r)
                    damp = 1.0 - jnp.exp((-_DAMP) * u * u * r)
                    coef = qd * damp * (-(_BOHR * _BOHR)) / (d * d * d)
                    fo = io3 + go * 3
                    vx = plsc.load_gather(vb, [fo])
                    vy = plsc.load_gather(vb, [fo + 1])
                    vz = plsc.load_gather(vb, [fo + 2])
                    exb[pl.ds(go, 16)] = vx * coef
                    eyb[pl.ds(go, 16)] = vy * coef
                    ezb[pl.ds(go, 16)] = vz * coef
                    return 0

                lax.fori_loop(0, 128 // 16, vbody, 0)

            for j in range(_G):
                sl = pl.ds(j * 128, 128)
                pltpu.async_copy(exb.at[sl], acc.at[is0.at[j]], ssc,
                                 add=True)
                pltpu.async_copy(eyb.at[sl], acc.at[is1.at[j]], ssc,
                                 add=True)
                pltpu.async_copy(ezb.at[sl], acc.at[is2.at[j]], ssc,
                                 add=True)
            for j in range(_G):
                sl = pl.ds(j * 128, 128)
                pltpu.make_async_copy(exb.at[sl], acc.at[is0.at[j]],
                                      ssc).wait()
                pltpu.make_async_copy(eyb.at[sl], acc.at[is1.at[j]],
                                      ssc).wait()
                pltpu.make_async_copy(ezb.at[sl], acc.at[is2.at[j]],
                                      ssc).wait()

        return 0

    lax.fori_loop(0, _TRIPS, chunk_body, 0)

    plsc.subcore_barrier()

    @pl.when(sid == 0)
    def _writeout():
        pltpu.sync_copy(acc, out_hbm.at[cid])


@functools.partial(
    pl.kernel,
    out_type=jax.ShapeDtypeStruct((_NC, 3 * _N_NODES), jnp.float32),
    mesh=plsc.VectorSubcoreMesh(core_axis_name="c", subcore_axis_name="s"),
    compiler_params=pltpu.CompilerParams(needs_layout_passes=False),
    scratch_types=[
        pltpu.VMEM_SHARED((3 * _N_NODES,), jnp.float32),
        pltpu.VMEM((_G, 128), jnp.int32),
        pltpu.VMEM((_G, 128), jnp.int32),
        pltpu.VMEM((_C,), jnp.float32),
        pltpu.VMEM((_C * 3,), jnp.float32),
        pltpu.VMEM((_C,), jnp.float32),
        pltpu.VMEM((_C,), jnp.float32),
        pltpu.VMEM((_C,), jnp.float32),
        pltpu.VMEM((_G, 128), jnp.int32),
        pltpu.VMEM((_G, 128), jnp.int32),
        pltpu.VMEM((_G, 128), jnp.int32),
        pltpu.VMEM((_C,), jnp.float32),
        pltpu.VMEM((_C,), jnp.float32),
        pltpu.VMEM((_C,), jnp.float32),
        pltpu.SemaphoreType.DMA,
        pltpu.SemaphoreType.DMA,
        pltpu.SemaphoreType.DMA,
        pltpu.SemaphoreType.DMA,
    ],
)
def _sc_field(src_hbm, dst_hbm, dist_hbm, vec_hbm, pv_hbm, q_hbm, zeros_hbm,
              out_hbm, acc, idxs, idxd, db, vb, psb, pdb, qdb,
              is0, is1, is2, exb, eyb, ezb, sidx, sdat, sgat, ssc):
    _sc_body(src_hbm, dst_hbm, dist_hbm, vec_hbm, pv_hbm, q_hbm, zeros_hbm,
             out_hbm, acc, idxs, idxd, db, vb, psb, pdb, qdb,
             is0, is1, is2, exb, eyb, ezb, sidx, sdat, sgat, ssc)


def kernel(species, edge_src, edge_dst, distances, vec, polarisability,
           charges):
    n = species.shape[0]

    padn = 784 * 128
    polp = jnp.pad(polarisability.astype(jnp.float32), (0, padn - n),
                   constant_values=1.0).reshape(784, 128)
    pv = pl.pallas_call(
        _pv_body,
        out_shape=jax.ShapeDtypeStruct((784, 128), jnp.float32),
    )(polp).reshape(-1)[:n]

    src2d = edge_src.astype(jnp.int32).reshape(-1, 128)
    dst2d = edge_dst.astype(jnp.int32).reshape(-1, 128)
    vecf = vec.astype(jnp.float32).reshape(-1)
    zeros = jnp.zeros((3 * n,), jnp.float32)

    partials = _sc_field(src2d, dst2d, distances.astype(jnp.float32), vecf,
                         pv, charges.astype(jnp.float32), zeros)

    padm = 2344 * 128
    flatp = jnp.pad(partials, ((0, 0), (0, padm - 3 * n))).reshape(
        2, 2344, 128)
    total = pl.pallas_call(
        _sum_body,
        out_shape=jax.ShapeDtypeStruct((2344, 128), jnp.float32),
    )(flatp)
    return total.reshape(-1)[:3 * n]


# speedup vs baseline: 1.0002x; 1.0002x over previous
import functools

import jax
import jax.numpy as jnp
from jax import lax
from jax.experimental import pallas as pl
from jax.experimental.pallas import tpu as pltpu
from jax.experimental.pallas import tpu_sc as plsc

_BOHR = 0.52917721067
_DAMP = 0.7
_NC = 2
_NS = 16
_NW = _NC * _NS
_C = 2048
_G = _C // 128

_N_NODES = 100000
_N_EDGES = 6400000
_NCHUNKS = _N_EDGES // _C
_TRIPS = -(-_NCHUNKS // _NW)


def _pv_body(pol_ref, out_ref):
    p = pol_ref[...] * (1.0 / _BOHR**3)
    out_ref[...] = jnp.exp(jnp.log(p) * (-1.0 / 6.0)) * (_BOHR**-0.5)


def _sum_body(a_ref, b_ref, out_ref):
    out_ref[...] = a_ref[...] + b_ref[...]


def _sc_body(src_hbm, dst_hbm, dist_hbm, vec_hbm, pv_hbm, q_hbm, zeros_hbm,
             out0_hbm, out1_hbm, acc, idxs, idxd, db, vb, psb, pdb, qdb,
             is0, is1, is2, exb, eyb, ezb, sidx, sdat, sgat, ssc):
    cid = lax.axis_index("c")
    sid = lax.axis_index("s")
    wid = sid * _NC + cid

    @pl.when(sid == 0)
    def _zero():
        pltpu.sync_copy(zeros_hbm, acc)

    plsc.subcore_barrier()

    io = lax.iota(jnp.int32, 16)
    io3 = io * 3

    def chunk_body(t, _):
        ch = wid + t * _NW

        @pl.when(ch < _NCHUNKS)
        def _do_chunk():
            base = ch * _C
            pltpu.async_copy(src_hbm.at[pl.ds(base, _C)], idxs, sidx)
            pltpu.async_copy(dst_hbm.at[pl.ds(base, _C)], idxd, sidx)
            pltpu.async_copy(dist_hbm.at[pl.ds(base, _C)], db, sdat)
            pltpu.async_copy(vec_hbm.at[pl.ds(base * 3, _C * 3)], vb, sdat)
            pltpu.make_async_copy(src_hbm.at[pl.ds(base, _C)], idxs,
                                  sidx).wait()
            pltpu.make_async_copy(dst_hbm.at[pl.ds(base, _C)], idxd,
                                  sidx).wait()
            for j in range(_G):
                sl = pl.ds(j * 128, 128)
                pltpu.async_copy(pv_hbm.at[idxs.at[sl]], psb.at[sl], sgat)
                pltpu.async_copy(pv_hbm.at[idxd.at[sl]], pdb.at[sl], sgat)
                pltpu.async_copy(q_hbm.at[idxd.at[sl]], qdb.at[sl], sgat)
            pltpu.make_async_copy(dist_hbm.at[pl.ds(base, _C)], db,
                                  sdat).wait()
            pltpu.make_async_copy(vec_hbm.at[pl.ds(base * 3, _C * 3)], vb,
                                  sdat).wait()
            for j in range(_G):
                sl = pl.ds(j * 128, 128)
                pltpu.make_async_copy(pv_hbm.at[idxs.at[sl]], psb.at[sl],
                                      sgat).wait()
                pltpu.make_async_copy(pv_hbm.at[idxd.at[sl]], pdb.at[sl],
                                      sgat).wait()
                pltpu.make_async_copy(q_hbm.at[idxd.at[sl]], qdb.at[sl],
                                      sgat).wait()

            for j in range(_G):
                ir0 = is0.at[j]
                ir1 = is1.at[j]
                ir2 = is2.at[j]

                def vbody(w, _, ir0=ir0, ir1=ir1, ir2=ir2, j=j):
                    o = w * 16
                    go = j * 128 + o
                    s16 = idxs[pl.ds(go, 16)]
                    s3 = s16 * 3
                    lanes = io + o
                    plsc.store_scatter(ir0, [lanes], s3)
                    plsc.store_scatter(ir1, [lanes], s3 + 1)
                    plsc.store_scatter(ir2, [lanes], s3 + 2)
                    d = db[pl.ds(go, 16)]
                    ps = psb[pl.ds(go, 16)]
                    pd = pdb[pl.ds(go, 16)]
                    qd = qdb[pl.ds(go, 16)]
                    u = d * ps * pd
                    bi = lax.bitcast_convert_type(u, jnp.int32)
                    r = lax.bitcast_convert_type(0x5F3759DF - (bi >> 1),
                                                 jnp.float32)
                    r = r * (1.5 - 0.5 * u * r * r)
                    r = r * (1.5 - 0.5 * u * r * r)
                    r = r * (1.5 - 0.5 * u * r * r)
                    damp = 1.0 - jnp.exp((-_DAMP) * u * u * r)
                    coef = qd * damp * (-(_BOHR * _BOHR)) / (d * d * d)
                    fo = io3 + go * 3
                    vx = plsc.load_gather(vb, [fo])
                    vy = plsc.load_gather(vb, [fo + 1])
                    vz = plsc.load_gather(vb, [fo + 2])
                    exb[pl.ds(go, 16)] = vx * coef
                    eyb[pl.ds(go, 16)] = vy * coef
                    ezb[pl.ds(go, 16)] = vz * coef
                    return 0

                lax.fori_loop(0, 128 // 16, vbody, 0)

            for j in range(_G):
                sl = pl.ds(j * 128, 128)
                pltpu.async_copy(exb.at[sl], acc.at[is0.at[j]], ssc,
                                 add=True)
                pltpu.async_copy(eyb.at[sl], acc.at[is1.at[j]], ssc,
                                 add=True)
                pltpu.async_copy(ezb.at[sl], acc.at[is2.at[j]], ssc,
                                 add=True)
            for j in range(_G):
                sl = pl.ds(j * 128, 128)
                pltpu.make_async_copy(exb.at[sl], acc.at[is0.at[j]],
                                      ssc).wait()
                pltpu.make_async_copy(eyb.at[sl], acc.at[is1.at[j]],
                                      ssc).wait()
                pltpu.make_async_copy(ezb.at[sl], acc.at[is2.at[j]],
                                      ssc).wait()

        return 0

    lax.fori_loop(0, _TRIPS, chunk_body, 0)

    plsc.subcore_barrier()

    @pl.when((sid == 0) & (cid == 0))
    def _writeout0():
        pltpu.sync_copy(acc, out0_hbm)

    @pl.when((sid == 0) & (cid == 1))
    def _writeout1():
        pltpu.sync_copy(acc, out1_hbm)


@functools.partial(
    pl.kernel,
    out_type=(jax.ShapeDtypeStruct((3 * _N_NODES,), jnp.float32),
              jax.ShapeDtypeStruct((3 * _N_NODES,), jnp.float32)),
    mesh=plsc.VectorSubcoreMesh(core_axis_name="c", subcore_axis_name="s"),
    compiler_params=pltpu.CompilerParams(needs_layout_passes=False),
    scratch_types=[
        pltpu.VMEM_SHARED((3 * _N_NODES,), jnp.float32),
        pltpu.VMEM((_C,), jnp.int32),
        pltpu.VMEM((_C,), jnp.int32),
        pltpu.VMEM((_C,), jnp.float32),
        pltpu.VMEM((_C * 3,), jnp.float32),
        pltpu.VMEM((_C,), jnp.float32),
        pltpu.VMEM((_C,), jnp.float32),
        pltpu.VMEM((_C,), jnp.float32),
        pltpu.VMEM((_G, 128), jnp.int32),
        pltpu.VMEM((_G, 128), jnp.int32),
        pltpu.VMEM((_G, 128), jnp.int32),
        pltpu.VMEM((_C,), jnp.float32),
        pltpu.VMEM((_C,), jnp.float32),
        pltpu.VMEM((_C,), jnp.float32),
        pltpu.SemaphoreType.DMA,
        pltpu.SemaphoreType.DMA,
        pltpu.SemaphoreType.DMA,
        pltpu.SemaphoreType.DMA,
    ],
)
def _sc_field(src_hbm, dst_hbm, dist_hbm, vec_hbm, pv_hbm, q_hbm, zeros_hbm,
              out0_hbm, out1_hbm, acc, idxs, idxd, db, vb, psb, pdb, qdb,
              is0, is1, is2, exb, eyb, ezb, sidx, sdat, sgat, ssc):
    _sc_body(src_hbm, dst_hbm, dist_hbm, vec_hbm, pv_hbm, q_hbm, zeros_hbm,
             out0_hbm, out1_hbm, acc, idxs, idxd, db, vb, psb, pdb, qdb,
             is0, is1, is2, exb, eyb, ezb, sidx, sdat, sgat, ssc)


def kernel(species, edge_src, edge_dst, distances, vec, polarisability,
           charges):
    n = species.shape[0]

    padn = 784 * 128
    polp = jnp.pad(polarisability.astype(jnp.float32), (0, padn - n),
                   constant_values=1.0).reshape(784, 128)
    pv = pl.pallas_call(
        _pv_body,
        out_shape=jax.ShapeDtypeStruct((784, 128), jnp.float32),
    )(polp).reshape(-1)[:n]

    vecf = vec.astype(jnp.float32).reshape(-1)
    zeros = jnp.zeros((3 * n,), jnp.float32)

    p0, p1 = _sc_field(edge_src.astype(jnp.int32),
                       edge_dst.astype(jnp.int32),
                       distances.astype(jnp.float32), vecf,
                       pv, charges.astype(jnp.float32), zeros)

    padm = 2344 * 128
    pada = jnp.pad(p0, (0, padm - 3 * n)).reshape(2344, 128)
    padb = jnp.pad(p1, (0, padm - 3 * n)).reshape(2344, 128)
    total = pl.pallas_call(
        _sum_body,
        out_shape=jax.ShapeDtypeStruct((2344, 128), jnp.float32),
    )(pada, padb)
    return total.reshape(-1)[:3 * n]


# speedup vs baseline: 5.7965x; 5.7953x over previous
import functools

import jax
import jax.numpy as jnp
from jax import lax
from jax.experimental import pallas as pl
from jax.experimental.pallas import tpu as pltpu
from jax.experimental.pallas import tpu_sc as plsc

_BOHR = 0.52917721067
_DAMP = 0.7
_NC = 2
_NS = 16
_NW = _NC * _NS
_C = 2048
_G = _C // 128

_N_NODES = 100000
_N_EDGES = 6400000
_NCHUNKS = _N_EDGES // _C
_TRIPS = -(-_NCHUNKS // _NW)


def _pv_body(pol_ref, out_ref):
    p = pol_ref[...] * (1.0 / _BOHR**3)
    out_ref[...] = jnp.exp(jnp.log(p) * (-1.0 / 6.0)) * (_BOHR**-0.5)


def _sum_body(a_ref, b_ref, out_ref):
    out_ref[...] = a_ref[...] + b_ref[...]


def _sc_body(src_hbm, dst_hbm, dist_hbm, vx_hbm, vy_hbm, vz_hbm, pv_hbm,
             q_hbm, zeros_hbm, out0_hbm, out1_hbm, acc, idxs, idxd, db,
             vxb, vyb, vzb, psb, pdb, qdb,
             is0, is1, is2, exb, eyb, ezb, sidx, sdat, sgat, ssc):
    cid = lax.axis_index("c")
    sid = lax.axis_index("s")
    wid = sid * _NC + cid

    @pl.when(sid == 0)
    def _zero():
        pltpu.sync_copy(zeros_hbm, acc)

    plsc.subcore_barrier()

    io = lax.iota(jnp.int32, 16)

    def chunk_body(t, _):
        ch = wid + t * _NW

        @pl.when(ch < _NCHUNKS)
        def _do_chunk():
            base = ch * _C
            pltpu.async_copy(src_hbm.at[pl.ds(base, _C)], idxs, sidx)
            pltpu.async_copy(dst_hbm.at[pl.ds(base, _C)], idxd, sidx)
            pltpu.async_copy(dist_hbm.at[pl.ds(base, _C)], db, sdat)
            pltpu.async_copy(vx_hbm.at[pl.ds(base, _C)], vxb, sdat)
            pltpu.async_copy(vy_hbm.at[pl.ds(base, _C)], vyb, sdat)
            pltpu.async_copy(vz_hbm.at[pl.ds(base, _C)], vzb, sdat)
            pltpu.make_async_copy(src_hbm.at[pl.ds(base, _C)], idxs,
                                  sidx).wait()
            pltpu.make_async_copy(dst_hbm.at[pl.ds(base, _C)], idxd,
                                  sidx).wait()
            for j in range(_G):
                sl = pl.ds(j * 128, 128)
                pltpu.async_copy(pv_hbm.at[idxs.at[sl]], psb.at[sl], sgat)
                pltpu.async_copy(pv_hbm.at[idxd.at[sl]], pdb.at[sl], sgat)
                pltpu.async_copy(q_hbm.at[idxd.at[sl]], qdb.at[sl], sgat)
            pltpu.make_async_copy(dist_hbm.at[pl.ds(base, _C)], db,
                                  sdat).wait()
            pltpu.make_async_copy(vx_hbm.at[pl.ds(base, _C)], vxb,
                                  sdat).wait()
            pltpu.make_async_copy(vy_hbm.at[pl.ds(base, _C)], vyb,
                                  sdat).wait()
            pltpu.make_async_copy(vz_hbm.at[pl.ds(base, _C)], vzb,
                                  sdat).wait()
            for j in range(_G):
                sl = pl.ds(j * 128, 128)
                pltpu.make_async_copy(pv_hbm.at[idxs.at[sl]], psb.at[sl],
                                      sgat).wait()
                pltpu.make_async_copy(pv_hbm.at[idxd.at[sl]], pdb.at[sl],
                                      sgat).wait()
                pltpu.make_async_copy(q_hbm.at[idxd.at[sl]], qdb.at[sl],
                                      sgat).wait()

            for j in range(_G):
                ir0 = is0.at[j]
                ir1 = is1.at[j]
                ir2 = is2.at[j]

                def vbody(w, _, ir0=ir0, ir1=ir1, ir2=ir2, j=j):
                    o = w * 16
                    go = j * 128 + o
                    s16 = idxs[pl.ds(go, 16)]
                    s3 = s16 * 3
                    lanes = io + o
                    plsc.store_scatter(ir0, [lanes], s3)
                    plsc.store_scatter(ir1, [lanes], s3 + 1)
                    plsc.store_scatter(ir2, [lanes], s3 + 2)
                    d = db[pl.ds(go, 16)]
                    ps = psb[pl.ds(go, 16)]
                    pd = pdb[pl.ds(go, 16)]
                    qd = qdb[pl.ds(go, 16)]
                    u = d * ps * pd
                    bi = lax.bitcast_convert_type(u, jnp.int32)
                    r = lax.bitcast_convert_type(0x5F3759DF - (bi >> 1),
                                                 jnp.float32)
                    r = r * (1.5 - 0.5 * u * r * r)
                    r = r * (1.5 - 0.5 * u * r * r)
                    r = r * (1.5 - 0.5 * u * r * r)
                    damp = 1.0 - jnp.exp((-_DAMP) * u * u * r)
                    coef = qd * damp * (-(_BOHR * _BOHR)) / (d * d * d)
                    exb[pl.ds(go, 16)] = vxb[pl.ds(go, 16)] * coef
                    eyb[pl.ds(go, 16)] = vyb[pl.ds(go, 16)] * coef
                    ezb[pl.ds(go, 16)] = vzb[pl.ds(go, 16)] * coef
                    return 0

                lax.fori_loop(0, 128 // 16, vbody, 0)

            for j in range(_G):
                sl = pl.ds(j * 128, 128)
                pltpu.async_copy(exb.at[sl], acc.at[is0.at[j]], ssc,
                                 add=True)
                pltpu.async_copy(eyb.at[sl], acc.at[is1.at[j]], ssc,
                                 add=True)
                pltpu.async_copy(ezb.at[sl], acc.at[is2.at[j]], ssc,
                                 add=True)
            for j in range(_G):
                sl = pl.ds(j * 128, 128)
                pltpu.make_async_copy(exb.at[sl], acc.at[is0.at[j]],
                                      ssc).wait()
                pltpu.make_async_copy(eyb.at[sl], acc.at[is1.at[j]],
                                      ssc).wait()
                pltpu.make_async_copy(ezb.at[sl], acc.at[is2.at[j]],
                                      ssc).wait()

        return 0

    lax.fori_loop(0, _TRIPS, chunk_body, 0)

    plsc.subcore_barrier()

    @pl.when((sid == 0) & (cid == 0))
    def _writeout0():
        pltpu.sync_copy(acc, out0_hbm)

    @pl.when((sid == 0) & (cid == 1))
    def _writeout1():
        pltpu.sync_copy(acc, out1_hbm)


@functools.partial(
    pl.kernel,
    out_type=(jax.ShapeDtypeStruct((3 * _N_NODES,), jnp.float32),
              jax.ShapeDtypeStruct((3 * _N_NODES,), jnp.float32)),
    mesh=plsc.VectorSubcoreMesh(core_axis_name="c", subcore_axis_name="s"),
    compiler_params=pltpu.CompilerParams(needs_layout_passes=False),
    scratch_types=[
        pltpu.VMEM_SHARED((3 * _N_NODES,), jnp.float32),
        pltpu.VMEM((_C,), jnp.int32),
        pltpu.VMEM((_C,), jnp.int32),
        pltpu.VMEM((_C,), jnp.float32),
        pltpu.VMEM((_C,), jnp.float32),
        pltpu.VMEM((_C,), jnp.float32),
        pltpu.VMEM((_C,), jnp.float32),
        pltpu.VMEM((_C,), jnp.float32),
        pltpu.VMEM((_C,), jnp.float32),
        pltpu.VMEM((_C,), jnp.float32),
        pltpu.VMEM((_G, 128), jnp.int32),
        pltpu.VMEM((_G, 128), jnp.int32),
        pltpu.VMEM((_G, 128), jnp.int32),
        pltpu.VMEM((_C,), jnp.float32),
        pltpu.VMEM((_C,), jnp.float32),
        pltpu.VMEM((_C,), jnp.float32),
        pltpu.SemaphoreType.DMA,
        pltpu.SemaphoreType.DMA,
        pltpu.SemaphoreType.DMA,
        pltpu.SemaphoreType.DMA,
    ],
)
def _sc_field(src_hbm, dst_hbm, dist_hbm, vx_hbm, vy_hbm, vz_hbm, pv_hbm,
              q_hbm, zeros_hbm, out0_hbm, out1_hbm, acc, idxs, idxd, db,
              vxb, vyb, vzb, psb, pdb, qdb,
              is0, is1, is2, exb, eyb, ezb, sidx, sdat, sgat, ssc):
    _sc_body(src_hbm, dst_hbm, dist_hbm, vx_hbm, vy_hbm, vz_hbm, pv_hbm,
             q_hbm, zeros_hbm, out0_hbm, out1_hbm, acc, idxs, idxd, db,
             vxb, vyb, vzb, psb, pdb, qdb,
             is0, is1, is2, exb, eyb, ezb, sidx, sdat, sgat, ssc)


def kernel(species, edge_src, edge_dst, distances, vec, polarisability,
           charges):
    n = species.shape[0]

    padn = 784 * 128
    polp = jnp.pad(polarisability.astype(jnp.float32), (0, padn - n),
                   constant_values=1.0).reshape(784, 128)
    pv = pl.pallas_call(
        _pv_body,
        out_shape=jax.ShapeDtypeStruct((784, 128), jnp.float32),
    )(polp).reshape(-1)[:n]

    vf = vec.astype(jnp.float32)
    vx, vy, vz = vf[:, 0], vf[:, 1], vf[:, 2]
    zeros = jnp.zeros((3 * n,), jnp.float32)

    p0, p1 = _sc_field(edge_src.astype(jnp.int32),
                       edge_dst.astype(jnp.int32),
                       distances.astype(jnp.float32), vx, vy, vz,
                       pv, charges.astype(jnp.float32), zeros)

    padm = 2344 * 128
    pada = jnp.pad(p0, (0, padm - 3 * n)).reshape(2344, 128)
    padb = jnp.pad(p1, (0, padm - 3 * n)).reshape(2344, 128)
    total = pl.pallas_call(
        _sum_body,
        out_shape=jax.ShapeDtypeStruct((2344, 128), jnp.float32),
    )(pada, padb)
    return total.reshape(-1)[:3 * n]
